# Initial kernel scaffold; baseline (speedup 1.0000x reference)
import jax
import jax.numpy as jnp
from jax.experimental import pallas as pl


def kernel(embedding0, w_relation, triplets):
    raise NotImplementedError("write your pallas kernel here")


# speedup vs baseline: 2.1849x; 2.1849x over previous
import functools

import jax
import jax.numpy as jnp
from jax import lax
from jax.experimental import pallas as pl
from jax.experimental.pallas import tpu as pltpu
from jax.experimental.pallas import tpu_sc as plsc

N_NODES = 10000
H_DIM = 128
NUM_RELS = 237
N_TRIPLETS = 320000

NC, NS, L = 2, 16, 16
NW = NC * NS
W_PER = N_TRIPLETS // NW
CHUNK = 80
NCHUNK = W_PER // CHUNK


def _sc_body(emb_hbm, wrel_hbm, src_hbm, rel_hbm, dst_hbm, out_hbm,
             src_v, rel_v, dst_v, s_v, r_v, o_v, out_v, sem):
    wid = lax.axis_index("s") * NC + lax.axis_index("c")
    wbase = wid * W_PER

    def chunk_body(g, carry):
        base = wbase + g * CHUNK
        pltpu.sync_copy(src_hbm.at[pl.ds(base, CHUNK)], src_v)
        pltpu.sync_copy(rel_hbm.at[pl.ds(base, CHUNK)], rel_v)
        pltpu.sync_copy(dst_hbm.at[pl.ds(base, CHUNK)], dst_v)
        c1 = pltpu.async_copy(emb_hbm.at[src_v], s_v, sem)
        c2 = pltpu.async_copy(wrel_hbm.at[rel_v], r_v, sem)
        c3 = pltpu.async_copy(emb_hbm.at[dst_v], o_v, sem)
        c1.wait()
        c2.wait()
        c3.wait()

        lanes = lax.iota(jnp.int32, L)

        def blk_body(tb, c):
            acc = jnp.zeros((L,), jnp.float32)
            for j in range(L):
                t = tb * L + j
                p = (s_v[t, pl.ds(0, L)] * r_v[t, pl.ds(0, L)]
                     * o_v[t, pl.ds(0, L)])
                for u in range(1, H_DIM // L):
                    p = p + (s_v[t, pl.ds(u * L, L)] * r_v[t, pl.ds(u * L, L)]
                             * o_v[t, pl.ds(u * L, L)])
                acc = jnp.where(lanes == j, jnp.sum(p), acc)
            out_v[pl.ds(tb * L, L)] = acc
            return c

        lax.fori_loop(0, CHUNK // L, blk_body, 0)

        pltpu.sync_copy(out_v, out_hbm.at[pl.ds(base, CHUNK)])
        return carry

    lax.fori_loop(0, NCHUNK, chunk_body, 0)


@jax.jit
def kernel(embedding0, w_relation, triplets):
    t = triplets.astype(jnp.int32)
    src = t[:, 0]
    rel = t[:, 1]
    dst = t[:, 2]
    mesh = plsc.VectorSubcoreMesh(core_axis_name="c", subcore_axis_name="s")
    k = pl.kernel(
        _sc_body,
        out_type=jax.ShapeDtypeStruct((N_TRIPLETS,), jnp.float32),
        mesh=mesh,
        compiler_params=pltpu.CompilerParams(needs_layout_passes=False),
        scratch_types=[
            pltpu.VMEM((CHUNK,), jnp.int32),
            pltpu.VMEM((CHUNK,), jnp.int32),
            pltpu.VMEM((CHUNK,), jnp.int32),
            pltpu.VMEM((CHUNK, H_DIM), jnp.float32),
            pltpu.VMEM((CHUNK, H_DIM), jnp.float32),
            pltpu.VMEM((CHUNK, H_DIM), jnp.float32),
            pltpu.VMEM((CHUNK,), jnp.float32),
            pltpu.SemaphoreType.DMA,
        ],
    )
    return k(embedding0, w_relation, src, rel, dst)


# speedup vs baseline: 4.6481x; 2.1273x over previous
import jax
import jax.numpy as jnp
from jax import lax
from jax.experimental import pallas as pl
from jax.experimental.pallas import tpu as pltpu
from jax.experimental.pallas import tpu_sc as plsc

N_NODES = 10000
H_DIM = 128
NUM_RELS = 237
N_TRIPLETS = 320000

NC, NS, L = 2, 16, 16
NW = NC * NS
W_PER = N_TRIPLETS // NW
CHUNK = 80
NCHUNK = W_PER // CHUNK
NGRP = H_DIM // L


def _sc_body(emb_hbm, wrel_hbm, src_hbm, rel_hbm, dst_hbm, out_hbm,
             src_all, rel_all, dst_all, wrel_v,
             s_v0, s_v1, o_v0, o_v1, out_v0, out_v1,
             sem_in0, sem_in1, sem_out0, sem_out1):
    s_v = (s_v0, s_v1)
    o_v = (o_v0, o_v1)
    out_v = (out_v0, out_v1)
    sem_in = (sem_in0, sem_in1)
    sem_out = (sem_out0, sem_out1)

    wid = lax.axis_index("s") * NC + lax.axis_index("c")
    wbase = wid * W_PER

    pltpu.sync_copy(src_hbm.at[pl.ds(wbase, W_PER)], src_all)
    pltpu.sync_copy(rel_hbm.at[pl.ds(wbase, W_PER)], rel_all)
    pltpu.sync_copy(dst_hbm.at[pl.ds(wbase, W_PER)], dst_all)
    pltpu.sync_copy(wrel_hbm, wrel_v)

    def gathers(g, b):
        off = g * CHUNK
        c1 = pltpu.make_async_copy(
            emb_hbm.at[src_all.at[pl.ds(off, CHUNK)]], s_v[b], sem_in[b])
        c2 = pltpu.make_async_copy(
            emb_hbm.at[dst_all.at[pl.ds(off, CHUNK)]], o_v[b], sem_in[b])
        return c1, c2

    def fire_in(g, b):
        c1, c2 = gathers(g, b)
        c1.start()
        c2.start()

    def wait_in(g, b):
        c1, c2 = gathers(g, b)
        c1.wait()
        c2.wait()

    def out_store(g, b):
        return pltpu.make_async_copy(
            out_v[b], out_hbm.at[pl.ds(wbase + g * CHUNK, CHUNK)], sem_out[b])

    lanes = lax.iota(jnp.int32, L)

    def compute(g, b):
        sb, ob = s_v[b], o_v[b]
        goff = g * CHUNK

        def blk_body(tb, c):
            relv = rel_all[pl.ds(goff + tb * L, L)]
            acc = jnp.zeros((L,), jnp.float32)
            for j in range(L):
                t = tb * L + j
                ridx = lax.squeeze(lax.slice(relv, (j,), (j + 1,)), (0,))
                p = (sb[t, pl.ds(0, L)] * wrel_v[ridx, pl.ds(0, L)]
                     * ob[t, pl.ds(0, L)])
                for u in range(1, NGRP):
                    p = p + (sb[t, pl.ds(u * L, L)]
                             * wrel_v[ridx, pl.ds(u * L, L)]
                             * ob[t, pl.ds(u * L, L)])
                acc = jnp.where(lanes == j, jnp.sum(p), acc)
            out_v[b][pl.ds(tb * L, L)] = acc
            return c

        lax.fori_loop(0, CHUNK // L, blk_body, 0)
        out_store(g, b).start()

    fire_in(0, 0)

    def loop_body(i, carry):
        for b in (0, 1):
            @pl.when(lax.rem(i, 2) == b)
            def _():
                @pl.when(i >= 2)
                def _():
                    out_store(i - 2, b).wait()
                wait_in(i, b)

                @pl.when(i + 1 < NCHUNK)
                def _():
                    fire_in(i + 1, 1 - b)
                compute(i, b)
        return carry

    lax.fori_loop(0, NCHUNK, loop_body, 0)
    out_store(NCHUNK - 2, (NCHUNK - 2) % 2).wait()
    out_store(NCHUNK - 1, (NCHUNK - 1) % 2).wait()


@jax.jit
def kernel(embedding0, w_relation, triplets):
    t = triplets.astype(jnp.int32)
    src = t[:, 0]
    rel = t[:, 1]
    dst = t[:, 2]
    mesh = plsc.VectorSubcoreMesh(core_axis_name="c", subcore_axis_name="s")
    k = pl.kernel(
        _sc_body,
        out_type=jax.ShapeDtypeStruct((N_TRIPLETS,), jnp.float32),
        mesh=mesh,
        compiler_params=pltpu.CompilerParams(needs_layout_passes=False),
        scratch_types=[
            pltpu.VMEM((W_PER,), jnp.int32),
            pltpu.VMEM((W_PER,), jnp.int32),
            pltpu.VMEM((W_PER,), jnp.int32),
            pltpu.VMEM((NUM_RELS, H_DIM), jnp.float32),
            pltpu.VMEM((CHUNK, H_DIM), jnp.float32),
            pltpu.VMEM((CHUNK, H_DIM), jnp.float32),
            pltpu.VMEM((CHUNK, H_DIM), jnp.float32),
            pltpu.VMEM((CHUNK, H_DIM), jnp.float32),
            pltpu.VMEM((CHUNK,), jnp.float32),
            pltpu.VMEM((CHUNK,), jnp.float32),
            pltpu.SemaphoreType.DMA,
            pltpu.SemaphoreType.DMA,
            pltpu.SemaphoreType.DMA,
            pltpu.SemaphoreType.DMA,
        ],
    )
    return k(embedding0, w_relation, src, rel, dst)
